# baseline (speedup 1.0000x reference)
import functools

import jax
import jax.numpy as jnp
from jax import lax
from jax.experimental import pallas as pl
from jax.experimental.pallas import tpu as pltpu
from jax.experimental.pallas import tpu_sc as plsc

NC = 2
NS = 16
NW = NC * NS
LANES = 16

BLK = 128
NBUF = 4
LOOKAHEAD = 3


def _make_kernel(B, L, V, H):
    assert B == NW * BLK
    assert H % LANES == 0
    NCH = L
    assert NCH % NBUF == 0
    G = NCH // NBUF

    mesh = plsc.VectorSubcoreMesh(
        core_axis_name="c", subcore_axis_name="s", num_cores=NC,
        num_subcores=NS)

    @functools.partial(
        pl.kernel,
        out_type=jax.ShapeDtypeStruct((L, H, B), jnp.float32),
        mesh=mesh,
        scratch_types=dict(
            idx_all=pltpu.VMEM((L, BLK), jnp.int32),
            pos_v=pltpu.VMEM((H, L), jnp.float32),
            rows=[pltpu.VMEM((BLK, H), jnp.float32) for _ in range(NBUF)],
            obuf=[pltpu.VMEM((H, BLK), jnp.float32) for _ in range(NBUF)],
            gsem=[pltpu.SemaphoreType.DMA for _ in range(NBUF)],
            ssem=[pltpu.SemaphoreType.DMA for _ in range(NBUF)],
        ),
        compiler_params=pltpu.CompilerParams(
            use_tc_tiling_on_sc=False, needs_layout_passes=False),
    )
    def emb_kernel(xt_hbm, emb_hbm, post_hbm, out_hbm, *, idx_all, pos_v,
                   rows, obuf, gsem, ssem):
        wid = lax.axis_index("s") * NC + lax.axis_index("c")
        b0 = wid * BLK

        pltpu.sync_copy(xt_hbm.at[:, pl.ds(b0, BLK)], idx_all)
        pltpu.sync_copy(post_hbm, pos_v)

        def gather_start(l, b):
            pltpu.async_copy(emb_hbm.at[idx_all.at[l]], rows[b], gsem[b])

        def gather_wait(l, b):
            pltpu.make_async_copy(
                emb_hbm.at[idx_all.at[l]], rows[b], gsem[b]).wait()

        def store_start(l, b):
            pltpu.async_copy(
                obuf[b], out_hbm.at[l, :, pl.ds(b0, BLK)], ssem[b])

        def store_wait(l, b):
            pltpu.make_async_copy(
                obuf[b], out_hbm.at[l, :, pl.ds(b0, BLK)], ssem[b]).wait()

        ridx = [jnp.int32(r0 * LANES) + lax.iota(jnp.int32, LANES)
                for r0 in range(BLK // LANES)]

        def transpose_add(l, b):
            lsplat = jnp.full((LANES,), l, dtype=jnp.int32)

            def hbody(h, carry):
                hsplat = jnp.full((LANES,), h, dtype=jnp.int32)
                pv = plsc.load_gather(pos_v, [hsplat, lsplat])
                for r0 in range(BLK // LANES):
                    v = plsc.load_gather(rows[b], [ridx[r0], hsplat])
                    obuf[b][h, pl.ds(r0 * LANES, LANES)] = v + pv
                return carry

            lax.fori_loop(0, H, hbody, 0, unroll=4)

        for b in range(LOOKAHEAD):
            gather_start(b, b)

        def group(g, carry):
            for b in range(NBUF):
                l = g * NBUF + b
                gq = l + LOOKAHEAD
                bg = (b + LOOKAHEAD) % NBUF

                @pl.when(gq < NCH)
                def _issue():
                    @pl.when(gq >= NBUF)
                    def _wait_store():
                        store_wait(gq - NBUF, bg)

                    gather_start(gq, bg)

                gather_wait(l, b)
                transpose_add(l, b)
                store_start(l, b)
            return carry

        lax.fori_loop(0, G, group, 0)

        for b in range(NBUF):
            store_wait(NCH - NBUF + b, b)

    return emb_kernel


def kernel(x, emb_table, pos_table):
    B, L = x.shape
    V, H = emb_table.shape
    xt = x.T.astype(jnp.int32)
    post = pos_table.T
    fn = _make_kernel(B, L, V, H)
    out = fn(xt, emb_table, post)
    return jnp.transpose(out, (2, 0, 1))


# speedup vs baseline: 1.6381x; 1.6381x over previous
import functools

import jax
import jax.numpy as jnp
from jax import lax
from jax.experimental import pallas as pl
from jax.experimental.pallas import tpu as pltpu
from jax.experimental.pallas import tpu_sc as plsc

NC = 2
NS = 16
NW = NC * NS
LANES = 16

BLK = 128
NBUF = 4
LOOKAHEAD = 3


def _make_kernel(B, L, V, H):
    assert B == NW * BLK
    assert H % LANES == 0
    NCH = L
    assert NCH % NBUF == 0
    G = NCH // NBUF

    mesh = plsc.VectorSubcoreMesh(
        core_axis_name="c", subcore_axis_name="s", num_cores=NC,
        num_subcores=NS)

    @functools.partial(
        pl.kernel,
        out_type=jax.ShapeDtypeStruct((L, H, B), jnp.float32),
        mesh=mesh,
        scratch_types=dict(
            idx_all=pltpu.VMEM((L, BLK), jnp.int32),
            pos_v=pltpu.VMEM((L, H), jnp.float32),
            rows=[pltpu.VMEM((BLK, H), jnp.float32) for _ in range(NBUF)],
            obuf=[pltpu.VMEM((H, BLK + 1), jnp.float32) for _ in range(NBUF)],
            gsem=[pltpu.SemaphoreType.DMA for _ in range(NBUF)],
            ssem=[pltpu.SemaphoreType.DMA for _ in range(NBUF)],
        ),
        compiler_params=pltpu.CompilerParams(
            use_tc_tiling_on_sc=False, needs_layout_passes=False),
    )
    def emb_kernel(xt_hbm, emb_hbm, pos_hbm, out_hbm, *, idx_all, pos_v,
                   rows, obuf, gsem, ssem):
        wid = lax.axis_index("s") * NC + lax.axis_index("c")
        b0 = wid * BLK

        pltpu.sync_copy(xt_hbm.at[:, pl.ds(b0, BLK)], idx_all)
        pltpu.sync_copy(pos_hbm, pos_v)

        def gather_start(l, b):
            pltpu.async_copy(emb_hbm.at[idx_all.at[l]], rows[b], gsem[b])

        def gather_wait(l, b):
            pltpu.make_async_copy(
                emb_hbm.at[idx_all.at[l]], rows[b], gsem[b]).wait()

        def store_start(l, b):
            pltpu.async_copy(obuf[b].at[:, pl.ds(0, BLK)],
                             out_hbm.at[l, :, pl.ds(b0, BLK)], ssem[b])

        def store_wait(l, b):
            pltpu.make_async_copy(
                obuf[b].at[:, pl.ds(0, BLK)],
                out_hbm.at[l, :, pl.ds(b0, BLK)], ssem[b]).wait()

        hidx = [jnp.int32(j * LANES) + lax.iota(jnp.int32, LANES)
                for j in range(H // LANES)]

        def transpose_add(l, b):
            pj = [pos_v[l, pl.ds(j * LANES, LANES)] for j in range(H // LANES)]

            def rbody(r, carry):
                rsplat = jnp.full((LANES,), r, dtype=jnp.int32)
                for j in range(H // LANES):
                    v = rows[b][r, pl.ds(j * LANES, LANES)]
                    plsc.store_scatter(obuf[b], [hidx[j], rsplat], v + pj[j])
                return carry

            lax.fori_loop(0, BLK, rbody, 0, unroll=4)

        for b in range(LOOKAHEAD):
            gather_start(b, b)

        def group(g, carry):
            for b in range(NBUF):
                l = g * NBUF + b
                gq = l + LOOKAHEAD
                bg = (b + LOOKAHEAD) % NBUF

                @pl.when(gq < NCH)
                def _issue():
                    @pl.when(gq >= NBUF)
                    def _wait_store():
                        store_wait(gq - NBUF, bg)

                    gather_start(gq, bg)

                gather_wait(l, b)
                transpose_add(l, b)
                store_start(l, b)
            return carry

        lax.fori_loop(0, G, group, 0)

        for b in range(NBUF):
            store_wait(NCH - NBUF + b, b)

    return emb_kernel


def kernel(x, emb_table, pos_table):
    B, L = x.shape
    V, H = emb_table.shape
    xt = x.T.astype(jnp.int32)
    fn = _make_kernel(B, L, V, H)
    out = fn(xt, emb_table, pos_table)
    return jnp.transpose(out, (2, 0, 1))
